# baseline (speedup 1.0000x reference)
import functools

import jax
import jax.numpy as jnp
from jax.experimental import pallas as pl
from jax.experimental.pallas import tpu as pltpu

N = 1024
H = 8
C = 16
FEAT = H * C
TJ = 256
NEG = -1e30


def _proj_kernel(x_ref, wl_ref, bl_ref, wr_ref, br_ref, att_ref,
                 xl_ref, xlt_ref, xr_ref, ut_ref, v_ref):
    x = x_ref[...]
    xl = jnp.dot(x, wl_ref[...], preferred_element_type=jnp.float32) + bl_ref[...]
    xr = jnp.dot(x, wr_ref[...], preferred_element_type=jnp.float32) + br_ref[...]
    xl_ref[...] = xl
    xlt_ref[...] = xl.T
    xr_ref[...] = xr
    xlt = xl.T
    ut_rows = []
    v_cols = []
    for h in range(H):
        u_row = jnp.zeros((1, N), jnp.float32)
        v_col = jnp.zeros((N, 1), jnp.float32)
        for c in range(C):
            f = h * C + c
            a6 = 0.6 * att_ref[h, c]
            u_row = u_row + a6 * xlt[f:f + 1, :]
            v_col = v_col + a6 * xr[:, f:f + 1]
        ut_rows.append(u_row)
        v_cols.append(v_col)
    ut_ref[...] = jnp.concatenate(ut_rows, axis=0)
    v_ref[...] = jnp.concatenate(v_cols, axis=1)


def _attn_kernel(xl_ref, xlt_ref, xr_ref, ut_ref, v_ref, adj_ref, att_ref,
                 bias_ref, out_ref, *, apply_elu):
    j0 = pl.program_id(0) * TJ
    adj_t = adj_ref[...].T
    row_j = jax.lax.broadcasted_iota(jnp.int32, (TJ, N), 0) + j0
    col_i = jax.lax.broadcasted_iota(jnp.int32, (TJ, N), 1)
    diag = row_j == col_i
    valid = jnp.logical_or(diag, adj_t != 0)

    outs = []
    for h in range(H):
        acc = v_ref[:, h:h + 1] + ut_ref[h:h + 1, :]
        for c in range(C):
            f = h * C + c
            a4 = 0.4 * att_ref[h, c]
            s = xr_ref[:, f:f + 1] + xlt_ref[f:f + 1, :]
            acc = acc + a4 * jnp.abs(s)
        acc = jnp.where(valid, acc, NEG)
        m = jnp.max(acc, axis=1, keepdims=True)
        p = jnp.exp(acc - m)
        den = jnp.sum(p, axis=1, keepdims=True) + 1e-16
        alpha = p / den
        agg = jnp.dot(alpha, xl_ref[:, h * C:(h + 1) * C],
                      preferred_element_type=jnp.float32)
        outs.append(agg)
    out = jnp.concatenate(outs, axis=1) + bias_ref[...]
    if apply_elu:
        out = jnp.where(out > 0, out, jnp.exp(jnp.minimum(out, 0.0)) - 1.0)
    out_ref[...] = out


def _project(x, wl, bl, wr, br, att):
    return pl.pallas_call(
        _proj_kernel,
        in_specs=[
            pl.BlockSpec((N, FEAT), lambda: (0, 0)),
            pl.BlockSpec((FEAT, FEAT), lambda: (0, 0)),
            pl.BlockSpec((1, FEAT), lambda: (0, 0)),
            pl.BlockSpec((FEAT, FEAT), lambda: (0, 0)),
            pl.BlockSpec((1, FEAT), lambda: (0, 0)),
            pl.BlockSpec(memory_space=pltpu.SMEM),
        ],
        out_shape=[
            jax.ShapeDtypeStruct((N, FEAT), jnp.float32),
            jax.ShapeDtypeStruct((FEAT, N), jnp.float32),
            jax.ShapeDtypeStruct((N, FEAT), jnp.float32),
            jax.ShapeDtypeStruct((H, N), jnp.float32),
            jax.ShapeDtypeStruct((N, H), jnp.float32),
        ],
    )(x, wl, bl.reshape(1, FEAT), wr, br.reshape(1, FEAT), att)


def _attention(xl, xlt, xr, ut, v, adj, att, bias, apply_elu):
    grid = (N // TJ,)
    return pl.pallas_call(
        functools.partial(_attn_kernel, apply_elu=apply_elu),
        grid=grid,
        in_specs=[
            pl.BlockSpec((N, FEAT), lambda j: (0, 0)),
            pl.BlockSpec((FEAT, N), lambda j: (0, 0)),
            pl.BlockSpec((TJ, FEAT), lambda j: (j, 0)),
            pl.BlockSpec((H, N), lambda j: (0, 0)),
            pl.BlockSpec((TJ, H), lambda j: (j, 0)),
            pl.BlockSpec((N, TJ), lambda j: (0, j)),
            pl.BlockSpec(memory_space=pltpu.SMEM),
            pl.BlockSpec((1, FEAT), lambda j: (0, 0)),
        ],
        out_specs=pl.BlockSpec((TJ, FEAT), lambda j: (j, 0)),
        out_shape=jax.ShapeDtypeStruct((N, FEAT), jnp.float32),
        compiler_params=pltpu.CompilerParams(
            dimension_semantics=("parallel",)),
    )(xl, xlt, xr, ut, v, adj, att, bias.reshape(1, FEAT))


def kernel(input, adj, Wl1, bl1, Wr1, br1, att1, bias1,
           Wl2, bl2, Wr2, br2, att2, bias2):
    b, n, ic, nf = input.shape
    x = input.reshape(n, ic * nf)
    adj32 = adj.astype(jnp.int32)
    xl1, xlt1, xr1, ut1, v1 = _project(x, Wl1, bl1, Wr1, br1, att1)
    h1 = _attention(xl1, xlt1, xr1, ut1, v1, adj32, att1, bias1,
                    apply_elu=False)
    xl2, xlt2, xr2, ut2, v2 = _project(h1, Wl2, bl2, Wr2, br2, att2)
    h2 = _attention(xl2, xlt2, xr2, ut2, v2, adj32, att2, bias2,
                    apply_elu=True)
    return h2.reshape(b, n, H * C)


# speedup vs baseline: 1.2565x; 1.2565x over previous
import functools

import jax
import jax.numpy as jnp
from jax.experimental import pallas as pl
from jax.experimental.pallas import tpu as pltpu

N = 1024
H = 8
C = 16
FEAT = H * C
TJ = 256
NEG = -1e30


def _proj_kernel(x_ref, wl_ref, bl_ref, wr_ref, br_ref, att_ref,
                 xl_ref, xlt_ref, xr_ref, ut_ref, vt_ref):
    x = x_ref[...]
    xl = jnp.dot(x, wl_ref[...], preferred_element_type=jnp.float32) + bl_ref[...]
    xr = jnp.dot(x, wr_ref[...], preferred_element_type=jnp.float32) + br_ref[...]
    xl_ref[...] = xl
    xlt_ref[...] = xl.T
    xr_ref[...] = xr
    xlt = xl.T
    xrt = xr.T
    ut_rows = []
    vt_rows = []
    for h in range(H):
        u_row = jnp.zeros((1, N), jnp.float32)
        v_row = jnp.zeros((1, N), jnp.float32)
        for c in range(C):
            f = h * C + c
            a6 = 0.6 * att_ref[h, c]
            u_row = u_row + a6 * xlt[f:f + 1, :]
            v_row = v_row + a6 * xrt[f:f + 1, :]
        ut_rows.append(u_row)
        vt_rows.append(v_row)
    ut_ref[...] = jnp.concatenate(ut_rows, axis=0)
    vt_ref[...] = jnp.concatenate(vt_rows, axis=0)


def _attn_kernel(xl_ref, xlt_ref, xr_ref, ut_ref, vt_ref, adj_ref, att_ref,
                 bias_ref, out_ref, *, apply_elu):
    j0 = pl.program_id(0) * TJ
    adj_t = adj_ref[...].T
    row_j = jax.lax.broadcasted_iota(jnp.int32, (TJ, N), 0) + j0
    col_i = jax.lax.broadcasted_iota(jnp.int32, (TJ, N), 1)
    diag = row_j == col_i
    valid = jnp.logical_or(diag, adj_t != 0)

    v_blk = vt_ref[...].T
    outs = []
    for h in range(H):
        acc = v_blk[:, h:h + 1] + ut_ref[h:h + 1, :]
        for c in range(C):
            f = h * C + c
            a4 = 0.4 * att_ref[h, c]
            s = xr_ref[:, f:f + 1] + xlt_ref[f:f + 1, :]
            acc = acc + a4 * jnp.abs(s)
        acc = jnp.where(valid, acc, NEG)
        m = jnp.max(acc, axis=1, keepdims=True)
        p = jnp.exp(acc - m)
        den = jnp.sum(p, axis=1, keepdims=True) + 1e-16
        alpha = p / den
        agg = jnp.dot(alpha, xl_ref[:, h * C:(h + 1) * C],
                      preferred_element_type=jnp.float32)
        outs.append(agg)
    out = jnp.concatenate(outs, axis=1) + bias_ref[...]
    if apply_elu:
        out = jnp.where(out > 0, out, jnp.exp(jnp.minimum(out, 0.0)) - 1.0)
    out_ref[...] = out


def _project(x, wl, bl, wr, br, att):
    return pl.pallas_call(
        _proj_kernel,
        in_specs=[
            pl.BlockSpec((N, FEAT), lambda: (0, 0)),
            pl.BlockSpec((FEAT, FEAT), lambda: (0, 0)),
            pl.BlockSpec((1, FEAT), lambda: (0, 0)),
            pl.BlockSpec((FEAT, FEAT), lambda: (0, 0)),
            pl.BlockSpec((1, FEAT), lambda: (0, 0)),
            pl.BlockSpec(memory_space=pltpu.SMEM),
        ],
        out_shape=[
            jax.ShapeDtypeStruct((N, FEAT), jnp.float32),
            jax.ShapeDtypeStruct((FEAT, N), jnp.float32),
            jax.ShapeDtypeStruct((N, FEAT), jnp.float32),
            jax.ShapeDtypeStruct((H, N), jnp.float32),
            jax.ShapeDtypeStruct((H, N), jnp.float32),
        ],
    )(x, wl, bl.reshape(1, FEAT), wr, br.reshape(1, FEAT), att)


def _attention(xl, xlt, xr, ut, v, adj, att, bias, apply_elu):
    grid = (N // TJ,)
    return pl.pallas_call(
        functools.partial(_attn_kernel, apply_elu=apply_elu),
        grid=grid,
        in_specs=[
            pl.BlockSpec((N, FEAT), lambda j: (0, 0)),
            pl.BlockSpec((FEAT, N), lambda j: (0, 0)),
            pl.BlockSpec((TJ, FEAT), lambda j: (j, 0)),
            pl.BlockSpec((H, N), lambda j: (0, 0)),
            pl.BlockSpec((H, TJ), lambda j: (0, j)),
            pl.BlockSpec((N, TJ), lambda j: (0, j)),
            pl.BlockSpec(memory_space=pltpu.SMEM),
            pl.BlockSpec((1, FEAT), lambda j: (0, 0)),
        ],
        out_specs=pl.BlockSpec((TJ, FEAT), lambda j: (j, 0)),
        out_shape=jax.ShapeDtypeStruct((N, FEAT), jnp.float32),
        compiler_params=pltpu.CompilerParams(
            dimension_semantics=("parallel",)),
    )(xl, xlt, xr, ut, v, adj, att, bias.reshape(1, FEAT))


def kernel(input, adj, Wl1, bl1, Wr1, br1, att1, bias1,
           Wl2, bl2, Wr2, br2, att2, bias2):
    b, n, ic, nf = input.shape
    x = input.reshape(n, ic * nf)
    adj32 = adj.astype(jnp.int32)
    xl1, xlt1, xr1, ut1, v1 = _project(x, Wl1, bl1, Wr1, br1, att1)
    h1 = _attention(xl1, xlt1, xr1, ut1, v1, adj32, att1, bias1,
                    apply_elu=False)
    xl2, xlt2, xr2, ut2, v2 = _project(h1, Wl2, bl2, Wr2, br2, att2)
    h2 = _attention(xl2, xlt2, xr2, ut2, v2, adj32, att2, bias2,
                    apply_elu=True)
    return h2.reshape(b, n, H * C)


# speedup vs baseline: 2.0134x; 1.6024x over previous
import functools

import jax
import jax.numpy as jnp
from jax.experimental import pallas as pl
from jax.experimental.pallas import tpu as pltpu

N = 1024
H = 8
C = 16
FEAT = H * C
TJ = 256
NEG = -1e30


def _proj_kernel(x_ref, wl_ref, bl_ref, wr_ref, br_ref, att_ref,
                 xl_ref, xlt_ref, xr_ref, ut_ref, vt_ref,
                 xlt_bf_ref, xr_bf_ref):
    x = x_ref[...]
    xl = jnp.dot(x, wl_ref[...], preferred_element_type=jnp.float32) + bl_ref[...]
    xr = jnp.dot(x, wr_ref[...], preferred_element_type=jnp.float32) + br_ref[...]
    xl_ref[...] = xl
    xlt_ref[...] = xl.T
    xr_ref[...] = xr
    xlt = xl.T
    xrt = xr.T
    ut_rows = []
    vt_rows = []
    for h in range(H):
        u_row = jnp.zeros((1, N), jnp.float32)
        v_row = jnp.zeros((1, N), jnp.float32)
        for c in range(C):
            f = h * C + c
            a6 = 0.6 * att_ref[h, c]
            u_row = u_row + a6 * xlt[f:f + 1, :]
            v_row = v_row + a6 * xrt[f:f + 1, :]
        ut_rows.append(u_row)
        vt_rows.append(v_row)
    ut_ref[...] = jnp.concatenate(ut_rows, axis=0)
    vt_ref[...] = jnp.concatenate(vt_rows, axis=0)
    xlt_bf_ref[...] = xlt.astype(jnp.bfloat16)
    xr_bf_ref[...] = xr.astype(jnp.bfloat16)


def _attn_kernel(xl_ref, xlt_bf_ref, xr_bf_ref, ut_ref, vt_ref, adj_ref,
                 att_ref, bias_ref, out_ref, *, apply_elu):
    j0 = pl.program_id(0) * TJ
    adj_t = adj_ref[...].T
    row_j = jax.lax.broadcasted_iota(jnp.int32, (TJ, N), 0) + j0
    col_i = jax.lax.broadcasted_iota(jnp.int32, (TJ, N), 1)
    diag = row_j == col_i
    valid = jnp.logical_or(diag, adj_t != 0)

    v_blk = vt_ref[...].T
    outs = []
    for h in range(H):
        acc_abs = jnp.zeros((TJ, N), jnp.bfloat16)
        for c in range(C):
            f = h * C + c
            a4 = (0.4 * att_ref[h, c]).astype(jnp.bfloat16)
            s = xr_bf_ref[:, f:f + 1] + xlt_bf_ref[f:f + 1, :]
            acc_abs = acc_abs + a4 * jnp.abs(s)
        acc = (v_blk[:, h:h + 1] + ut_ref[h:h + 1, :]
               + acc_abs.astype(jnp.float32))
        acc = jnp.where(valid, acc, NEG)
        m = jnp.max(acc, axis=1, keepdims=True)
        p = jnp.exp(acc - m)
        den = jnp.sum(p, axis=1, keepdims=True) + 1e-16
        alpha = p / den
        agg = jnp.dot(alpha, xl_ref[:, h * C:(h + 1) * C],
                      preferred_element_type=jnp.float32)
        outs.append(agg)
    out = jnp.concatenate(outs, axis=1) + bias_ref[...]
    if apply_elu:
        out = jnp.where(out > 0, out, jnp.exp(jnp.minimum(out, 0.0)) - 1.0)
    out_ref[...] = out


def _project(x, wl, bl, wr, br, att):
    return pl.pallas_call(
        _proj_kernel,
        in_specs=[
            pl.BlockSpec((N, FEAT), lambda: (0, 0)),
            pl.BlockSpec((FEAT, FEAT), lambda: (0, 0)),
            pl.BlockSpec((1, FEAT), lambda: (0, 0)),
            pl.BlockSpec((FEAT, FEAT), lambda: (0, 0)),
            pl.BlockSpec((1, FEAT), lambda: (0, 0)),
            pl.BlockSpec(memory_space=pltpu.SMEM),
        ],
        out_shape=[
            jax.ShapeDtypeStruct((N, FEAT), jnp.float32),
            jax.ShapeDtypeStruct((FEAT, N), jnp.float32),
            jax.ShapeDtypeStruct((N, FEAT), jnp.float32),
            jax.ShapeDtypeStruct((H, N), jnp.float32),
            jax.ShapeDtypeStruct((H, N), jnp.float32),
            jax.ShapeDtypeStruct((FEAT, N), jnp.bfloat16),
            jax.ShapeDtypeStruct((N, FEAT), jnp.bfloat16),
        ],
    )(x, wl, bl.reshape(1, FEAT), wr, br.reshape(1, FEAT), att)


def _attention(xl, xlt_bf, xr_bf, ut, vt, adj, att, bias, apply_elu):
    grid = (N // TJ,)
    return pl.pallas_call(
        functools.partial(_attn_kernel, apply_elu=apply_elu),
        grid=grid,
        in_specs=[
            pl.BlockSpec((N, FEAT), lambda j: (0, 0)),
            pl.BlockSpec((FEAT, N), lambda j: (0, 0)),
            pl.BlockSpec((TJ, FEAT), lambda j: (j, 0)),
            pl.BlockSpec((H, N), lambda j: (0, 0)),
            pl.BlockSpec((H, TJ), lambda j: (0, j)),
            pl.BlockSpec((N, TJ), lambda j: (0, j)),
            pl.BlockSpec(memory_space=pltpu.SMEM),
            pl.BlockSpec((1, FEAT), lambda j: (0, 0)),
        ],
        out_specs=pl.BlockSpec((TJ, FEAT), lambda j: (j, 0)),
        out_shape=jax.ShapeDtypeStruct((N, FEAT), jnp.float32),
        compiler_params=pltpu.CompilerParams(
            dimension_semantics=("parallel",)),
    )(xl, xlt_bf, xr_bf, ut, vt, adj, att, bias.reshape(1, FEAT))


def kernel(input, adj, Wl1, bl1, Wr1, br1, att1, bias1,
           Wl2, bl2, Wr2, br2, att2, bias2):
    b, n, ic, nf = input.shape
    x = input.reshape(n, ic * nf)
    adj32 = adj.astype(jnp.int32)
    xl1, _, _, ut1, vt1, xltb1, xrb1 = _project(x, Wl1, bl1, Wr1, br1, att1)
    h1 = _attention(xl1, xltb1, xrb1, ut1, vt1, adj32, att1, bias1,
                    apply_elu=False)
    xl2, _, _, ut2, vt2, xltb2, xrb2 = _project(h1, Wl2, bl2, Wr2, br2, att2)
    h2 = _attention(xl2, xltb2, xrb2, ut2, vt2, adj32, att2, bias2,
                    apply_elu=True)
    return h2.reshape(b, n, H * C)


# speedup vs baseline: 2.0240x; 1.0053x over previous
import jax
import jax.numpy as jnp
from jax.experimental import pallas as pl
from jax.experimental.pallas import tpu as pltpu

N = 1024
H = 8
C = 16
FEAT = H * C
TJ = 256
NEG = -1e30


def _fused_kernel(x_ref, adj_ref, wl_ref, bl_ref, wr_ref, br_ref, att_ref,
                  bias_ref, out_ref,
                  xl_s, xltb_s, xrb_s, ut_s, vt_s, h1_s):
    l = pl.program_id(0)
    j = pl.program_id(1)
    j0 = j * TJ

    @pl.when(j == 0)
    def _proj():
        x = jnp.where(l == 0, x_ref[...], h1_s[...])
        xl = jnp.dot(x, wl_ref[0], preferred_element_type=jnp.float32) \
            + bl_ref[0]
        xr = jnp.dot(x, wr_ref[0], preferred_element_type=jnp.float32) \
            + br_ref[0]
        xlt = xl.T
        xrt = xr.T
        ut_rows = []
        vt_rows = []
        for h in range(H):
            u_row = jnp.zeros((1, N), jnp.float32)
            v_row = jnp.zeros((1, N), jnp.float32)
            for c in range(C):
                f = h * C + c
                a6 = 0.6 * att_ref[0, h, c]
                u_row = u_row + a6 * xlt[f:f + 1, :]
                v_row = v_row + a6 * xrt[f:f + 1, :]
            ut_rows.append(u_row)
            vt_rows.append(v_row)
        xl_s[...] = xl
        xltb_s[...] = xlt.astype(jnp.bfloat16)
        xrb_s[...] = xr.astype(jnp.bfloat16)
        ut_s[...] = jnp.concatenate(ut_rows, axis=0)
        vt_s[...] = jnp.concatenate(vt_rows, axis=0)

    adj_t = adj_ref[...].T
    row_j = jax.lax.broadcasted_iota(jnp.int32, (TJ, N), 0) + j0
    col_i = jax.lax.broadcasted_iota(jnp.int32, (TJ, N), 1)
    valid = jnp.logical_or(row_j == col_i, adj_t != 0)

    v_blk = vt_s[:, pl.ds(j0, TJ)].T
    outs = []
    for h in range(H):
        acc_abs = jnp.zeros((TJ, N), jnp.bfloat16)
        for c in range(C):
            f = h * C + c
            a4 = (0.4 * att_ref[0, h, c]).astype(jnp.bfloat16)
            s = xrb_s[pl.ds(j0, TJ), f:f + 1] + xltb_s[f:f + 1, :]
            acc_abs = acc_abs + a4 * jnp.abs(s)
        acc = (v_blk[:, h:h + 1] + ut_s[h:h + 1, :]
               + acc_abs.astype(jnp.float32))
        acc = jnp.where(valid, acc, NEG)
        m = jnp.max(acc, axis=1, keepdims=True)
        p = jnp.exp(acc - m)
        den = jnp.sum(p, axis=1, keepdims=True) + 1e-16
        alpha = p / den
        agg = jnp.dot(alpha, xl_s[:, h * C:(h + 1) * C],
                      preferred_element_type=jnp.float32)
        outs.append(agg)
    out = jnp.concatenate(outs, axis=1) + bias_ref[0]

    @pl.when(l == 0)
    def _store_h1():
        h1_s[pl.ds(j0, TJ), :] = out
        out_ref[...] = out

    @pl.when(l == 1)
    def _store_out():
        out_ref[...] = jnp.where(
            out > 0, out, jnp.exp(jnp.minimum(out, 0.0)) - 1.0)


def kernel(input, adj, Wl1, bl1, Wr1, br1, att1, bias1,
           Wl2, bl2, Wr2, br2, att2, bias2):
    b, n, ic, nf = input.shape
    x = input.reshape(n, ic * nf)
    adj32 = adj.astype(jnp.int32)
    wl = jnp.stack([Wl1, Wl2])
    wr = jnp.stack([Wr1, Wr2])
    blv = jnp.stack([bl1.reshape(1, FEAT), bl2.reshape(1, FEAT)])
    brv = jnp.stack([br1.reshape(1, FEAT), br2.reshape(1, FEAT)])
    att = jnp.stack([att1, att2])
    biasv = jnp.stack([bias1.reshape(1, FEAT), bias2.reshape(1, FEAT)])

    h2 = pl.pallas_call(
        _fused_kernel,
        grid=(2, N // TJ),
        in_specs=[
            pl.BlockSpec((N, FEAT), lambda l, j: (0, 0)),
            pl.BlockSpec((N, TJ), lambda l, j: (0, j)),
            pl.BlockSpec((1, FEAT, FEAT), lambda l, j: (l, 0, 0)),
            pl.BlockSpec((1, 1, FEAT), lambda l, j: (l, 0, 0)),
            pl.BlockSpec((1, FEAT, FEAT), lambda l, j: (l, 0, 0)),
            pl.BlockSpec((1, 1, FEAT), lambda l, j: (l, 0, 0)),
            pl.BlockSpec((1, H, C), lambda l, j: (l, 0, 0),
                         memory_space=pltpu.SMEM),
            pl.BlockSpec((1, 1, FEAT), lambda l, j: (l, 0, 0)),
        ],
        out_specs=pl.BlockSpec((TJ, FEAT), lambda l, j: (j, 0)),
        out_shape=jax.ShapeDtypeStruct((N, FEAT), jnp.float32),
        scratch_shapes=[
            pltpu.VMEM((N, FEAT), jnp.float32),
            pltpu.VMEM((FEAT, N), jnp.bfloat16),
            pltpu.VMEM((N, FEAT), jnp.bfloat16),
            pltpu.VMEM((H, N), jnp.float32),
            pltpu.VMEM((H, N), jnp.float32),
            pltpu.VMEM((N, FEAT), jnp.float32),
        ],
        compiler_params=pltpu.CompilerParams(
            dimension_semantics=("arbitrary", "arbitrary")),
    )(x, adj32, wl, blv, wr, brv, att, biasv)
    return h2.reshape(b, n, H * C)


# speedup vs baseline: 2.1980x; 1.0860x over previous
import jax
import jax.numpy as jnp
from jax.experimental import pallas as pl
from jax.experimental.pallas import tpu as pltpu

N = 1024
H = 8
C = 16
FEAT = H * C
TJ = 512
NEG = -1e30


def _fused_kernel(x_ref, adj_ref, wl_ref, bl_ref, wr_ref, br_ref, att_ref,
                  bias_ref, out_ref,
                  xl_s, xltb_s, xrb_s, ut_s, vt_s, h1_s):
    l = pl.program_id(0)
    j = pl.program_id(1)
    j0 = j * TJ

    @pl.when(j == 0)
    def _proj():
        x = jnp.where(l == 0, x_ref[...], h1_s[...])
        xl = jnp.dot(x, wl_ref[0], preferred_element_type=jnp.float32) \
            + bl_ref[0]
        xr = jnp.dot(x, wr_ref[0], preferred_element_type=jnp.float32) \
            + br_ref[0]
        xlt = xl.T
        xrt = xr.T
        ut_rows = []
        vt_rows = []
        for h in range(H):
            u_row = jnp.zeros((1, N), jnp.float32)
            v_row = jnp.zeros((1, N), jnp.float32)
            for c in range(C):
                f = h * C + c
                a6 = 0.6 * att_ref[0, h, c]
                u_row = u_row + a6 * xlt[f:f + 1, :]
                v_row = v_row + a6 * xrt[f:f + 1, :]
            ut_rows.append(u_row)
            vt_rows.append(v_row)
        xl_s[...] = xl
        xltb_s[...] = xlt.astype(jnp.bfloat16)
        xrb_s[...] = xr.astype(jnp.bfloat16)
        ut_s[...] = jnp.concatenate(ut_rows, axis=0)
        vt_s[...] = jnp.concatenate(vt_rows, axis=0)

    adj_t = adj_ref[...].T
    row_j = jax.lax.broadcasted_iota(jnp.int32, (TJ, N), 0) + j0
    col_i = jax.lax.broadcasted_iota(jnp.int32, (TJ, N), 1)
    valid = jnp.logical_or(row_j == col_i, adj_t != 0)

    v_blk = vt_s[:, pl.ds(j0, TJ)].T
    outs = []
    for h in range(H):
        acc_abs = jnp.zeros((TJ, N), jnp.bfloat16)
        for c in range(C):
            f = h * C + c
            a4 = (0.4 * att_ref[0, h, c]).astype(jnp.bfloat16)
            s = xrb_s[pl.ds(j0, TJ), f:f + 1] + xltb_s[f:f + 1, :]
            acc_abs = acc_abs + a4 * jnp.abs(s)
        acc = (v_blk[:, h:h + 1] + ut_s[h:h + 1, :]
               + acc_abs.astype(jnp.float32))
        acc = jnp.where(valid, acc, NEG)
        m = jnp.max(acc, axis=1, keepdims=True)
        p = jnp.exp(acc - m)
        den = jnp.sum(p, axis=1, keepdims=True) + 1e-16
        agg = jnp.dot(p, xl_s[:, h * C:(h + 1) * C],
                      preferred_element_type=jnp.float32)
        outs.append(agg / den)
    out = jnp.concatenate(outs, axis=1) + bias_ref[0]

    @pl.when(l == 0)
    def _store_h1():
        h1_s[pl.ds(j0, TJ), :] = out
        out_ref[...] = out

    @pl.when(l == 1)
    def _store_out():
        out_ref[...] = jnp.where(
            out > 0, out, jnp.exp(jnp.minimum(out, 0.0)) - 1.0)


def kernel(input, adj, Wl1, bl1, Wr1, br1, att1, bias1,
           Wl2, bl2, Wr2, br2, att2, bias2):
    b, n, ic, nf = input.shape
    x = input.reshape(n, ic * nf)
    adj32 = adj.astype(jnp.int32)
    wl = jnp.stack([Wl1, Wl2])
    wr = jnp.stack([Wr1, Wr2])
    blv = jnp.stack([bl1.reshape(1, FEAT), bl2.reshape(1, FEAT)])
    brv = jnp.stack([br1.reshape(1, FEAT), br2.reshape(1, FEAT)])
    att = jnp.stack([att1, att2])
    biasv = jnp.stack([bias1.reshape(1, FEAT), bias2.reshape(1, FEAT)])

    h2 = pl.pallas_call(
        _fused_kernel,
        grid=(2, N // TJ),
        in_specs=[
            pl.BlockSpec((N, FEAT), lambda l, j: (0, 0)),
            pl.BlockSpec((N, TJ), lambda l, j: (0, j)),
            pl.BlockSpec((1, FEAT, FEAT), lambda l, j: (l, 0, 0)),
            pl.BlockSpec((1, 1, FEAT), lambda l, j: (l, 0, 0)),
            pl.BlockSpec((1, FEAT, FEAT), lambda l, j: (l, 0, 0)),
            pl.BlockSpec((1, 1, FEAT), lambda l, j: (l, 0, 0)),
            pl.BlockSpec((1, H, C), lambda l, j: (l, 0, 0),
                         memory_space=pltpu.SMEM),
            pl.BlockSpec((1, 1, FEAT), lambda l, j: (l, 0, 0)),
        ],
        out_specs=pl.BlockSpec((TJ, FEAT), lambda l, j: (j, 0)),
        out_shape=jax.ShapeDtypeStruct((N, FEAT), jnp.float32),
        scratch_shapes=[
            pltpu.VMEM((N, FEAT), jnp.float32),
            pltpu.VMEM((FEAT, N), jnp.bfloat16),
            pltpu.VMEM((N, FEAT), jnp.bfloat16),
            pltpu.VMEM((H, N), jnp.float32),
            pltpu.VMEM((H, N), jnp.float32),
            pltpu.VMEM((N, FEAT), jnp.float32),
        ],
        compiler_params=pltpu.CompilerParams(
            dimension_semantics=("arbitrary", "arbitrary")),
    )(x, adj32, wl, blv, wr, brv, att, biasv)
    return h2.reshape(b, n, H * C)


# speedup vs baseline: 2.3647x; 1.0758x over previous
import jax
import jax.numpy as jnp
from jax.experimental import pallas as pl
from jax.experimental.pallas import tpu as pltpu

N = 1024
H = 8
C = 16
FEAT = H * C
TJ = 512
NEG = -1e30


def _fused_kernel(x_ref, adj_ref, wl1_ref, bl1_ref, wr1_ref, br1_ref,
                  att1_ref, bias1_ref, wl2_ref, bl2_ref, wr2_ref, br2_ref,
                  att2_ref, bias2_ref, out_ref,
                  xl_s, xltb_s, xrb_s, ut_s, vt_s, h1_s):
    l = pl.program_id(0)
    j = pl.program_id(1)
    j0 = j * TJ

    def _proj(x, wl_ref, bl_ref, wr_ref, br_ref, att_ref):
        xl = jnp.dot(x, wl_ref[...], preferred_element_type=jnp.float32) \
            + bl_ref[...]
        xr = jnp.dot(x, wr_ref[...], preferred_element_type=jnp.float32) \
            + br_ref[...]
        xlt = xl.T
        xrt = xr.T
        ut_rows = []
        vt_rows = []
        for h in range(H):
            u_row = jnp.zeros((1, N), jnp.float32)
            v_row = jnp.zeros((1, N), jnp.float32)
            for c in range(C):
                f = h * C + c
                a6 = 0.6 * att_ref[h, c]
                u_row = u_row + a6 * xlt[f:f + 1, :]
                v_row = v_row + a6 * xrt[f:f + 1, :]
            ut_rows.append(u_row)
            vt_rows.append(v_row)
        xl_s[...] = xl
        xltb_s[...] = xlt.astype(jnp.bfloat16)
        xrb_s[...] = xr.astype(jnp.bfloat16)
        ut_s[...] = jnp.concatenate(ut_rows, axis=0)
        vt_s[...] = jnp.concatenate(vt_rows, axis=0)

    @pl.when(jnp.logical_and(l == 0, j == 0))
    def _proj1():
        _proj(x_ref[...], wl1_ref, bl1_ref, wr1_ref, br1_ref, att1_ref)

    @pl.when(jnp.logical_and(l == 1, j == 0))
    def _proj2():
        _proj(h1_s[...], wl2_ref, bl2_ref, wr2_ref, br2_ref, att2_ref)

    adj_t = adj_ref[...].T
    row_j = jax.lax.broadcasted_iota(jnp.int32, (TJ, N), 0) + j0
    col_i = jax.lax.broadcasted_iota(jnp.int32, (TJ, N), 1)
    valid = jnp.logical_or(row_j == col_i, adj_t != 0)

    v_blk = vt_s[:, pl.ds(j0, TJ)].T
    outs = []
    for h in range(H):
        acc_abs = None
        for c in range(C):
            f = h * C + c
            a1 = 0.4 * att1_ref[h, c]
            a2 = 0.4 * att2_ref[h, c]
            a4 = jnp.where(l == 0, a1, a2).astype(jnp.bfloat16)
            s = xrb_s[pl.ds(j0, TJ), f:f + 1] + xltb_s[f:f + 1, :]
            t = a4 * jnp.abs(s)
            acc_abs = t if acc_abs is None else acc_abs + t
        acc_abs = jnp.where(valid, acc_abs, jnp.bfloat16(NEG))
        acc = (v_blk[:, h:h + 1] + ut_s[h:h + 1, :]
               + acc_abs.astype(jnp.float32))
        m = jnp.max(acc, axis=1, keepdims=True)
        p = jnp.exp(acc - m)
        den = jnp.sum(p, axis=1, keepdims=True) + 1e-16
        agg = jnp.dot(p, xl_s[:, h * C:(h + 1) * C],
                      preferred_element_type=jnp.float32)
        outs.append(agg / den)
    bias = jnp.where(l == 0, bias1_ref[...], bias2_ref[...])
    out = jnp.concatenate(outs, axis=1) + bias

    @pl.when(l == 0)
    def _store_h1():
        h1_s[pl.ds(j0, TJ), :] = out
        out_ref[...] = out

    @pl.when(l == 1)
    def _store_out():
        out_ref[...] = jnp.where(
            out > 0, out, jnp.exp(jnp.minimum(out, 0.0)) - 1.0)


def kernel(input, adj, Wl1, bl1, Wr1, br1, att1, bias1,
           Wl2, bl2, Wr2, br2, att2, bias2):
    b, n, ic, nf = input.shape
    x = input.reshape(n, ic * nf)
    adj32 = adj.astype(jnp.int32)
    full = pl.BlockSpec((FEAT, FEAT), lambda l, j: (0, 0))
    brow = pl.BlockSpec((1, FEAT), lambda l, j: (0, 0))
    smem = pl.BlockSpec(memory_space=pltpu.SMEM)

    h2 = pl.pallas_call(
        _fused_kernel,
        grid=(2, N // TJ),
        in_specs=[
            pl.BlockSpec((N, FEAT), lambda l, j: (0, 0)),
            pl.BlockSpec((N, TJ), lambda l, j: (0, j)),
            full, brow, full, brow, smem, brow,
            full, brow, full, brow, smem, brow,
        ],
        out_specs=pl.BlockSpec((TJ, FEAT), lambda l, j: (j, 0)),
        out_shape=jax.ShapeDtypeStruct((N, FEAT), jnp.float32),
        scratch_shapes=[
            pltpu.VMEM((N, FEAT), jnp.float32),
            pltpu.VMEM((FEAT, N), jnp.bfloat16),
            pltpu.VMEM((N, FEAT), jnp.bfloat16),
            pltpu.VMEM((H, N), jnp.float32),
            pltpu.VMEM((H, N), jnp.float32),
            pltpu.VMEM((N, FEAT), jnp.float32),
        ],
        compiler_params=pltpu.CompilerParams(
            dimension_semantics=("arbitrary", "arbitrary")),
    )(x, adj32,
      Wl1, bl1.reshape(1, FEAT), Wr1, br1.reshape(1, FEAT), att1,
      bias1.reshape(1, FEAT),
      Wl2, bl2.reshape(1, FEAT), Wr2, br2.reshape(1, FEAT), att2,
      bias2.reshape(1, FEAT))
    return h2.reshape(b, n, H * C)
